# baseline (speedup 1.0000x reference)
import jax
import jax.numpy as jnp
from jax.experimental import pallas as pl

L = 2


def _ssp(x):
    return jax.nn.softplus(x) - jnp.log(2.0)


def _ident(x_ref, o_ref):
    o_ref[...] = x_ref[...]


def kernel(positions, atom_types, idx_i, idx_j, seg_i, emb, centers, gamma,
           Wb1, bb1, Wc1, bc1, Wc2, bc2, Wb2, bb2, Wb3, bb3, Wd, bd, We):
    Nat = positions.shape[1]

    def total_e(pos, types):
        feats = jnp.take(emb, types, axis=0)
        disp = pos[idx_i] - pos[idx_j]
        dist = jnp.sqrt(jnp.sum(disp ** 2, axis=-1) + 1e-08)
        rbf = jnp.exp(-gamma * (dist[:, None] - centers) ** 2)
        for l in range(L):
            x = feats @ Wb1[l] + bb1[l]
            filt = _ssp(rbf @ Wc1[l] + bc1[l])
            filt = _ssp(filt @ Wc2[l] + bc2[l])
            msg = jnp.take(x, idx_j, axis=0) * filt
            agg = jax.ops.segment_sum(msg, seg_i, num_segments=Nat)
            h = _ssp(agg @ Wb2[l] + bb2[l])
            h = h @ Wb3[l] + bb3[l]
            feats = feats + h
        h = _ssp(feats @ Wd + bd)
        energy = h @ We
        return jnp.sum(energy, axis=0)[0]

    forces = -jax.vmap(jax.grad(total_e))(positions, atom_types)
    out = forces[:, :1, :]
    flat = jnp.pad(out.reshape(3), (0, 125)).reshape(1, 128)
    flat = pl.pallas_call(
        _ident, out_shape=jax.ShapeDtypeStruct((1, 128), jnp.float32))(flat)
    return flat[0, :3].reshape(1, 1, 3)


# speedup vs baseline: 3.4701x; 3.4701x over previous
import functools
import jax
import jax.numpy as jnp
from jax import lax
from jax.experimental import pallas as pl
import jax.experimental.pallas.tpu as pltpu
from jax.experimental.pallas import tpu_sc as plsc

NAT = 10000
NE = 160000
D = 128
RB = 32
NW = 32
EW = 5008
E2 = NW * EW
PADE = E2 - NE
CAP = 5120
EC = NW * CAP
CH = 128
ZR = 624
NROW = NAT // 16
BLKN = 2000
BLKE = 2048
BLKF = 1024
LOG2 = 0.6931471805599453


def _ssp(x):
    return jnp.logaddexp(x, 0.0) - LOG2


def _sig(x):
    return jax.nn.sigmoid(x)


def _mesh():
    return plsc.VectorSubcoreMesh(core_axis_name="c", subcore_axis_name="s")


def _sc_distcomp(pos, ii0, jj0):
    def body(pos_hbm, ii_hbm, jj_hbm,
             ss_hbm, counts_hbm, cii_hbm, cjj_hbm, css_hbm,
             pos_v, ii_v, jj_v, ss_v, cii_v, cjj_v, css_v, sem):
        ci = lax.axis_index("c")
        si = lax.axis_index("s")
        w = ci * 16 + si
        base = w * EW
        pltpu.sync_copy(pos_hbm, pos_v)
        pltpu.sync_copy(ii_hbm.at[pl.ds(base, EW)], ii_v)
        pltpu.sync_copy(jj_hbm.at[pl.ds(base, EW)], jj_v)
        lanes = lax.broadcasted_iota(jnp.int32, (16,), 0)
        c1 = jnp.full((16,), 1, jnp.int32)
        c2 = jnp.full((16,), 2, jnp.int32)

        def step(t, cnt):
            ii = ii_v[pl.ds(t * 16, 16)]
            jj = jj_v[pl.ds(t * 16, 16)]
            fi = ii * 3
            fj = jj * 3
            xi = plsc.load_gather(pos_v, [fi])
            yi = plsc.load_gather(pos_v, [fi + c1])
            zi = plsc.load_gather(pos_v, [fi + c2])
            xj = plsc.load_gather(pos_v, [fj])
            yj = plsc.load_gather(pos_v, [fj + c1])
            zj = plsc.load_gather(pos_v, [fj + c2])
            dx = xi - xj
            dy = yi - yj
            dz = zi - zj
            ss = dx * dx + dy * dy + dz * dz
            ss_v[pl.ds(t * 16, 16)] = ss
            gid = base + t * 16 + lanes
            m = ((ii == 0) | (jj == 0)) & (gid < NE)
            plsc.store_compressed(cii_v.at[pl.ds(cnt, 16)], ii, mask=m)
            plsc.store_compressed(cjj_v.at[pl.ds(cnt, 16)], jj, mask=m)
            plsc.store_compressed(css_v.at[pl.ds(cnt, 16)], ss, mask=m)
            npos = jnp.max(plsc.all_reduce_population_count(m))
            return cnt + npos

        cnt = lax.fori_loop(0, EW // 16, step, jnp.int32(0))
        cii_v[pl.ds(cnt, 16)] = jnp.zeros((16,), jnp.int32)
        cjj_v[pl.ds(cnt, 16)] = jnp.zeros((16,), jnp.int32)
        css_v[pl.ds(cnt, 16)] = jnp.zeros((16,), jnp.float32)
        pltpu.sync_copy(ss_v, ss_hbm.at[pl.ds(base, EW)])
        ii_v[pl.ds(0, 16)] = c1 * cnt
        pltpu.sync_copy(ii_v.at[pl.ds(0, 16)],
                        counts_hbm.at[pl.ds(w * 16, 16)])
        wb = pl.multiple_of(w * CAP, 16)
        pltpu.sync_copy(cii_v, cii_hbm.at[pl.ds(wb, CAP)])
        pltpu.sync_copy(cjj_v, cjj_hbm.at[pl.ds(wb, CAP)])
        pltpu.sync_copy(css_v, css_hbm.at[pl.ds(wb, CAP)])

    f = pl.kernel(
        body,
        out_type=[
            jax.ShapeDtypeStruct((E2,), jnp.float32),
            jax.ShapeDtypeStruct((NW * 16,), jnp.int32),
            jax.ShapeDtypeStruct((NW * CAP,), jnp.int32),
            jax.ShapeDtypeStruct((NW * CAP,), jnp.int32),
            jax.ShapeDtypeStruct((NW * CAP,), jnp.float32),
        ],
        mesh=_mesh(),
        compiler_params=pltpu.CompilerParams(needs_layout_passes=False),
        scratch_types=[
            pltpu.VMEM((NAT * 3,), jnp.float32),
            pltpu.VMEM((EW,), jnp.int32),
            pltpu.VMEM((EW,), jnp.int32),
            pltpu.VMEM((EW,), jnp.float32),
            pltpu.VMEM((CAP,), jnp.int32),
            pltpu.VMEM((CAP,), jnp.int32),
            pltpu.VMEM((CAP,), jnp.float32),
            pltpu.SemaphoreType.DMA,
        ],
    )
    return f(pos.reshape(NAT * 3), ii0, jj0)


def _sc_gms(tab, gidx, sidx, filt):
    def body(tab_hbm, gi_hbm, si_hbm, filt_hbm, aggp_hbm,
             gi_v, si_v, gi_t, si_t, rows_v, filt_v, agg_sh, sem):
        ci = lax.axis_index("c")
        si_ax = lax.axis_index("s")
        w = ci * 16 + si_ax
        base = w * EW

        def zstep(i, _):
            r = lax.shift_right_logical(i, 3)
            k = (i & 7) * 16
            rows_v[r, pl.ds(k, 16)] = jnp.zeros((16,), jnp.float32)
            return 0

        lax.fori_loop(0, CH * 8, zstep, 0)
        r0 = pl.multiple_of(si_ax * ZR, 16)
        pltpu.sync_copy(rows_v, agg_sh.at[pl.ds(r0, CH), :])
        pltpu.sync_copy(rows_v, agg_sh.at[pl.ds(r0 + CH, CH), :])
        pltpu.sync_copy(rows_v, agg_sh.at[pl.ds(r0 + 2 * CH, CH), :])
        pltpu.sync_copy(rows_v, agg_sh.at[pl.ds(r0 + 3 * CH, CH), :])
        pltpu.sync_copy(rows_v.at[pl.ds(0, 112), :],
                        agg_sh.at[pl.ds(r0 + 4 * CH, 112), :])
        pltpu.sync_copy(rows_v.at[pl.ds(0, 16), :],
                        agg_sh.at[pl.ds(16 * ZR, 16), :])
        plsc.subcore_barrier()

        def mul_loop(n8, rv, fv):
            def mstep(i, _):
                r = lax.shift_right_logical(i, 3)
                k = (i & 7) * 16
                rv[r, pl.ds(k, 16)] = rv[r, pl.ds(k, 16)] * fv[r, pl.ds(k, 16)]
                return 0
            lax.fori_loop(0, n8, mstep, 0)

        def chunk(t, _):
            bc = pl.multiple_of(base + t * CH, 16)
            pltpu.sync_copy(gi_hbm.at[pl.ds(bc, CH)], gi_v)
            pltpu.sync_copy(si_hbm.at[pl.ds(bc, CH)], si_v)
            pltpu.async_copy(tab_hbm.at[gi_v], rows_v, sem).wait()
            pltpu.sync_copy(filt_hbm.at[pl.ds(bc, CH), :], filt_v)
            mul_loop(CH * 8, rows_v, filt_v)
            pltpu.sync_copy(rows_v, agg_sh.at[si_v], add=True)
            return 0

        lax.fori_loop(0, 39, chunk, 0)
        bt = base + 39 * CH
        pltpu.sync_copy(gi_hbm.at[pl.ds(bt, 16)], gi_t)
        pltpu.sync_copy(si_hbm.at[pl.ds(bt, 16)], si_t)
        pltpu.async_copy(tab_hbm.at[gi_t], rows_v.at[pl.ds(0, 16), :],
                         sem).wait()
        pltpu.sync_copy(filt_hbm.at[pl.ds(bt, 16), :],
                        filt_v.at[pl.ds(0, 16), :])
        mul_loop(16 * 8, rows_v, filt_v)
        pltpu.sync_copy(rows_v.at[pl.ds(0, 16), :], agg_sh.at[si_t], add=True)
        plsc.subcore_barrier()
        pltpu.sync_copy(agg_sh.at[pl.ds(r0, CH), :],
                        aggp_hbm.at[ci, pl.ds(r0, CH), :])
        pltpu.sync_copy(agg_sh.at[pl.ds(r0 + CH, CH), :],
                        aggp_hbm.at[ci, pl.ds(r0 + CH, CH), :])
        pltpu.sync_copy(agg_sh.at[pl.ds(r0 + 2 * CH, CH), :],
                        aggp_hbm.at[ci, pl.ds(r0 + 2 * CH, CH), :])
        pltpu.sync_copy(agg_sh.at[pl.ds(r0 + 3 * CH, CH), :],
                        aggp_hbm.at[ci, pl.ds(r0 + 3 * CH, CH), :])
        pltpu.sync_copy(agg_sh.at[pl.ds(r0 + 4 * CH, 112), :],
                        aggp_hbm.at[ci, pl.ds(r0 + 4 * CH, 112), :])
        pltpu.sync_copy(agg_sh.at[pl.ds(16 * ZR, 16), :],
                        aggp_hbm.at[ci, pl.ds(16 * ZR, 16), :])

    f = pl.kernel(
        body,
        out_type=jax.ShapeDtypeStruct((2, NAT, D), jnp.float32),
        mesh=_mesh(),
        compiler_params=pltpu.CompilerParams(needs_layout_passes=False),
        scratch_types=[
            pltpu.VMEM((CH,), jnp.int32),
            pltpu.VMEM((CH,), jnp.int32),
            pltpu.VMEM((16,), jnp.int32),
            pltpu.VMEM((16,), jnp.int32),
            pltpu.VMEM((CH, D), jnp.float32),
            pltpu.VMEM((CH, D), jnp.float32),
            pltpu.VMEM_SHARED((NAT + 16, D), jnp.float32),
            pltpu.SemaphoreType.DMA,
        ],
    )
    return f(tab, gidx, sidx, filt)


def _sc_gats(counts, cii, cjj, css, pos, x0, x1, dg0, dg1):
    def body(counts_hbm, cii_hbm, cjj_hbm, css_hbm, pos_hbm,
             x0_hbm, x1_hbm, dg0_hbm, dg1_hbm,
             cnt_hbm, df0_hbm, df1_hbm,
             pxi_hbm, pyi_hbm, pzi_hbm, pxj_hbm, pyj_hbm, pzj_hbm,
             ssv_hbm, mi_hbm, mj_hbm, vv_hbm,
             counts_v, pos_v, ii_t, jj_t, ssv_t,
             gx0, gx1, gd0, gd1, d0_v, d1_v, stage_v, stage_f, sem):
        ci = lax.axis_index("c")
        si = lax.axis_index("s")
        w = ci * 16 + si
        pltpu.sync_copy(counts_hbm, counts_v)
        pltpu.sync_copy(pos_hbm, pos_v)
        lanes = lax.broadcasted_iota(jnp.int32, (16,), 0)
        c1 = jnp.full((16,), 1, jnp.int32)
        c2 = jnp.full((16,), 2, jnp.int32)

        def acc(v, carry):
            off, mycnt, tot = carry
            cv = jnp.max(counts_v[pl.ds(v * 16, 16)])
            cp = (cv + 15) & (-16)
            off = off + jnp.where(v < w, cp, 0)
            mycnt = jnp.where(v == w, cv, mycnt)
            return (off, mycnt, tot + cp)

        off, mycnt, tot = lax.fori_loop(
            0, NW, acc, (jnp.int32(0), jnp.int32(0), jnp.int32(0)))
        stage_v[pl.ds(0, 16)] = jnp.full((16,), 1, jnp.int32) * tot
        pltpu.sync_copy(stage_v, cnt_hbm)
        nst = lax.shift_right_logical(mycnt + 15, 4)

        def fout(val, dst, orow):
            stage_f[pl.ds(0, 16)] = val
            pltpu.sync_copy(stage_f, dst.at[pl.ds(orow, 16)])

        def step(t, _):
            rb = pl.multiple_of(w * CAP + t * 16, 16)
            pltpu.sync_copy(cii_hbm.at[pl.ds(rb, 16)], ii_t)
            pltpu.sync_copy(cjj_hbm.at[pl.ds(rb, 16)], jj_t)
            pltpu.sync_copy(css_hbm.at[pl.ds(rb, 16)], ssv_t)
            pltpu.async_copy(x0_hbm.at[jj_t], gx0, sem).wait()
            pltpu.async_copy(x1_hbm.at[jj_t], gx1, sem).wait()
            pltpu.async_copy(dg0_hbm.at[ii_t], gd0, sem).wait()
            pltpu.async_copy(dg1_hbm.at[ii_t], gd1, sem).wait()

            def mstep(i, _):
                r = lax.shift_right_logical(i, 3)
                k = (i & 7) * 16
                d0_v[r, pl.ds(k, 16)] = (gx0[r, pl.ds(k, 16)]
                                         * gd0[r, pl.ds(k, 16)])
                d1_v[r, pl.ds(k, 16)] = (gx1[r, pl.ds(k, 16)]
                                         * gd1[r, pl.ds(k, 16)])
                return 0

            lax.fori_loop(0, 128, mstep, 0)
            orow = pl.multiple_of(off + t * 16, 16)
            pltpu.sync_copy(d0_v, df0_hbm.at[pl.ds(orow, 16), :])
            pltpu.sync_copy(d1_v, df1_hbm.at[pl.ds(orow, 16), :])
            ii = ii_t[...]
            jj = jj_t[...]
            one = jnp.full((16,), 1.0, jnp.float32)
            zero = jnp.zeros((16,), jnp.float32)
            fi = ii * 3
            fj = jj * 3
            fout(plsc.load_gather(pos_v, [fi]), pxi_hbm, orow)
            fout(plsc.load_gather(pos_v, [fi + c1]), pyi_hbm, orow)
            fout(plsc.load_gather(pos_v, [fi + c2]), pzi_hbm, orow)
            fout(plsc.load_gather(pos_v, [fj]), pxj_hbm, orow)
            fout(plsc.load_gather(pos_v, [fj + c1]), pyj_hbm, orow)
            fout(plsc.load_gather(pos_v, [fj + c2]), pzj_hbm, orow)
            fout(ssv_t[...], ssv_hbm, orow)
            fout(jnp.where(ii == 0, one, zero), mi_hbm, orow)
            fout(jnp.where(jj == 0, one, zero), mj_hbm, orow)
            fout(jnp.where(t * 16 + lanes < mycnt, one, zero), vv_hbm, orow)
            return 0

        lax.fori_loop(0, nst, step, 0)

    fld = jax.ShapeDtypeStruct((EC,), jnp.float32)
    f = pl.kernel(
        body,
        out_type=[
            jax.ShapeDtypeStruct((16,), jnp.int32),
            jax.ShapeDtypeStruct((EC, D), jnp.float32),
            jax.ShapeDtypeStruct((EC, D), jnp.float32),
            fld, fld, fld, fld, fld, fld, fld, fld, fld, fld,
        ],
        mesh=_mesh(),
        compiler_params=pltpu.CompilerParams(needs_layout_passes=False),
        scratch_types=[
            pltpu.VMEM((NW * 16,), jnp.int32),
            pltpu.VMEM((NAT * 3,), jnp.float32),
            pltpu.VMEM((16,), jnp.int32),
            pltpu.VMEM((16,), jnp.int32),
            pltpu.VMEM((16,), jnp.float32),
            pltpu.VMEM((16, D), jnp.float32),
            pltpu.VMEM((16, D), jnp.float32),
            pltpu.VMEM((16, D), jnp.float32),
            pltpu.VMEM((16, D), jnp.float32),
            pltpu.VMEM((16, D), jnp.float32),
            pltpu.VMEM((16, D), jnp.float32),
            pltpu.VMEM((16,), jnp.int32),
            pltpu.VMEM((16,), jnp.float32),
            pltpu.SemaphoreType.DMA,
        ],
    )
    return f(counts, cii, cjj, css, pos.reshape(NAT * 3), x0, x1, dg0, dg1)


def _tc_embed(types, embp, w1, b1):
    def body(t_ref, e_ref, w_ref, b_ref, f_ref, x_ref):
        oh = (t_ref[0, 0, :][:, None]
              == lax.broadcasted_iota(jnp.int32, (BLKN, 16), 1))
        f = oh.astype(jnp.float32) @ e_ref[...]
        f_ref[...] = f
        x_ref[...] = f @ w_ref[...] + b_ref[...][None, :]

    types = types.reshape(NAT // BLKN, 1, BLKN)
    return pl.pallas_call(
        body,
        grid=(NAT // BLKN,),
        in_specs=[
            pl.BlockSpec((1, 1, BLKN), lambda i: (i, 0, 0)),
            pl.BlockSpec((16, D), lambda i: (0, 0)),
            pl.BlockSpec((D, D), lambda i: (0, 0)),
            pl.BlockSpec((D,), lambda i: (0,)),
        ],
        out_specs=[
            pl.BlockSpec((BLKN, D), lambda i: (i, 0)),
            pl.BlockSpec((BLKN, D), lambda i: (i, 0)),
        ],
        out_shape=[
            jax.ShapeDtypeStruct((NAT, D), jnp.float32),
            jax.ShapeDtypeStruct((NAT, D), jnp.float32),
        ],
    )(types, embp, w1, b1)


def _tc_filt(ss, w1, b1, w2, b2, centers, gamma):
    def body(ss_ref, w1_ref, b1_ref, w2_ref, b2_ref, c_ref, g_ref, o_ref):
        dist = jnp.sqrt(ss_ref[...] + 1e-08)
        rbf = jnp.exp(-g_ref[...][None, :]
                      * (dist[:, None] - c_ref[...][None, :]) ** 2)
        a1 = rbf @ w1_ref[...] + b1_ref[...][None, :]
        a2 = _ssp(a1) @ w2_ref[...] + b2_ref[...][None, :]
        o_ref[...] = _ssp(a2)

    nblk = (E2 + BLKE - 1) // BLKE
    return pl.pallas_call(
        body,
        grid=(nblk,),
        in_specs=[
            pl.BlockSpec((BLKE,), lambda i: (i,)),
            pl.BlockSpec((RB, D), lambda i: (0, 0)),
            pl.BlockSpec((D,), lambda i: (0,)),
            pl.BlockSpec((D, D), lambda i: (0, 0)),
            pl.BlockSpec((D,), lambda i: (0,)),
            pl.BlockSpec((RB,), lambda i: (0,)),
            pl.BlockSpec((RB,), lambda i: (0,)),
        ],
        out_specs=pl.BlockSpec((BLKE, D), lambda i: (i, 0)),
        out_shape=jax.ShapeDtypeStruct((E2, D), jnp.float32),
    )(ss, w1, b1, w2, b2, centers, gamma)


def _node_specs(n_in, n_out, extra_w):
    in_specs = [pl.BlockSpec((BLKN, D), lambda i: (i, 0)) for _ in range(n_in)]
    in_specs += extra_w
    out_specs = [pl.BlockSpec((BLKN, D), lambda i: (i, 0))
                 for _ in range(n_out)]
    out_shape = [jax.ShapeDtypeStruct((NAT, D), jnp.float32)
                 for _ in range(n_out)]
    return in_specs, out_specs, out_shape


_WSPEC = pl.BlockSpec((D, D), lambda i: (0, 0))
_BSPEC = pl.BlockSpec((D,), lambda i: (0,))
_PSPEC = pl.BlockSpec((2, BLKN, D), lambda i: (0, i, 0))


def _tc_mid0(aggp, feats0, w2, b2, w3, b3, w1n, b1n):
    def body(a_ref, f_ref, w2r, b2r, w3r, b3r, w1r, b1r,
             f1_ref, hp_ref, x1_ref):
        agg = a_ref[0] + a_ref[1]
        hpre = agg @ w2r[...] + b2r[...][None, :]
        hp_ref[...] = hpre
        f1 = f_ref[...] + _ssp(hpre) @ w3r[...] + b3r[...][None, :]
        f1_ref[...] = f1
        x1_ref[...] = f1 @ w1r[...] + b1r[...][None, :]

    in_specs = [_PSPEC, pl.BlockSpec((BLKN, D), lambda i: (i, 0)),
                _WSPEC, _BSPEC, _WSPEC, _BSPEC, _WSPEC, _BSPEC]
    _, out_specs, out_shape = _node_specs(0, 3, [])
    return pl.pallas_call(
        body, grid=(NAT // BLKN,), in_specs=in_specs,
        out_specs=out_specs, out_shape=out_shape,
    )(aggp, feats0, w2, b2, w3, b3, w1n, b1n)


def _tc_mid1(aggp, feats1, w2, b2, w3, b3):
    def body(a_ref, f_ref, w2r, b2r, w3r, b3r, f2_ref, hp_ref):
        agg = a_ref[0] + a_ref[1]
        hpre = agg @ w2r[...] + b2r[...][None, :]
        hp_ref[...] = hpre
        f2_ref[...] = f_ref[...] + _ssp(hpre) @ w3r[...] + b3r[...][None, :]

    in_specs = [_PSPEC, pl.BlockSpec((BLKN, D), lambda i: (i, 0)),
                _WSPEC, _BSPEC, _WSPEC, _BSPEC]
    _, out_specs, out_shape = _node_specs(0, 2, [])
    return pl.pallas_call(
        body, grid=(NAT // BLKN,), in_specs=in_specs,
        out_specs=out_specs, out_shape=out_shape,
    )(aggp, feats1, w2, b2, w3, b3)


def _tc_bwd1(feats2, hpre1, Wd, bd, We, w3, w2):
    def body(f_ref, hp_ref, wd_ref, bd_ref, we_ref, w3r, w2r,
             df_ref, da_ref):
        z = f_ref[...] @ wd_ref[...] + bd_ref[...][None, :]
        df2 = lax.dot_general(
            _sig(z) * we_ref[...].reshape(1, D // 2), wd_ref[...],
            (((1,), (1,)), ((), ())))
        df_ref[...] = df2
        t = lax.dot_general(df2, w3r[...], (((1,), (1,)), ((), ())))
        da_ref[...] = lax.dot_general(
            t * _sig(hp_ref[...]), w2r[...], (((1,), (1,)), ((), ())))

    in_specs = [pl.BlockSpec((BLKN, D), lambda i: (i, 0)),
                pl.BlockSpec((BLKN, D), lambda i: (i, 0)),
                pl.BlockSpec((D, D // 2), lambda i: (0, 0)),
                pl.BlockSpec((D // 2,), lambda i: (0,)),
                pl.BlockSpec((D // 2, 1), lambda i: (0, 0)),
                _WSPEC, _WSPEC]
    _, out_specs, out_shape = _node_specs(0, 2, [])
    return pl.pallas_call(
        body, grid=(NAT // BLKN,), in_specs=in_specs,
        out_specs=out_specs, out_shape=out_shape,
    )(feats2, hpre1, Wd, bd, We, w3, w2)


def _tc_bwd0(dfeats2, dxp, w1n, hpre0, w2, w3):
    def body(df_ref, dx_ref, w1r, hp_ref, w2r, w3r, da_ref):
        dx1 = dx_ref[0] + dx_ref[1]
        df1 = df_ref[...] + lax.dot_general(
            dx1, w1r[...], (((1,), (1,)), ((), ())))
        t = lax.dot_general(df1, w3r[...], (((1,), (1,)), ((), ())))
        da_ref[...] = lax.dot_general(
            t * _sig(hp_ref[...]), w2r[...], (((1,), (1,)), ((), ())))

    in_specs = [pl.BlockSpec((BLKN, D), lambda i: (i, 0)), _PSPEC,
                _WSPEC, pl.BlockSpec((BLKN, D), lambda i: (i, 0)),
                _WSPEC, _WSPEC]
    _, out_specs, out_shape = _node_specs(0, 1, [])
    return pl.pallas_call(
        body, grid=(NAT // BLKN,), in_specs=in_specs,
        out_specs=out_specs, out_shape=out_shape,
    )(dfeats2, dxp, w1n, hpre0, w2, w3)[0]


def _tc_force(cnt, df0, df1, flds, w10, b10, w20, b20,
              w11, b11, w21, b21, centers, gamma, p0pad):
    def body(cnt_ref, df0_ref, df1_ref,
             pxi_r, pyi_r, pzi_r, pxj_r, pyj_r, pzj_r, ss_r, mi_r, mj_r, vv_r,
             w10r, b10r, w20r, b20r, w11r, b11r, w21r, b21r,
             c_ref, gm_ref, p0_ref, o_ref):
        i = pl.program_id(0)

        @pl.when(i == 0)
        def _init():
            o_ref[...] = jnp.zeros((8, 128), jnp.float32)

        @pl.when(i * BLKF < cnt_ref[0])
        def _work():
            ss = ss_r[...]
            dist = jnp.sqrt(ss + 1e-08)
            cen = c_ref[...][None, :]
            gam = gm_ref[...][None, :]
            diff = dist[:, None] - cen
            rbf = jnp.exp(-gam * diff ** 2)
            drbf = jnp.zeros((BLKF, RB), jnp.float32)
            for (w1r, b1r, w2r, b2r, dfr) in (
                    (w10r, b10r, w20r, b20r, df0_ref),
                    (w11r, b11r, w21r, b21r, df1_ref)):
                a1 = rbf @ w1r[...] + b1r[...][None, :]
                a2 = _ssp(a1) @ w2r[...] + b2r[...][None, :]
                da2 = dfr[...] * _sig(a2)
                da1 = lax.dot_general(
                    da2, w2r[...], (((1,), (1,)), ((), ()))) * _sig(a1)
                drbf = drbf + lax.dot_general(
                    da1, w1r[...], (((1,), (1,)), ((), ())))
            dd = jnp.sum(drbf * rbf * (-2.0 * gam * diff), axis=1)
            wgt = vv_r[...] * dd / dist
            p0x = p0_ref[0, 0]
            p0y = p0_ref[0, 1]
            p0z = p0_ref[0, 2]
            mi = mi_r[...]
            mj = mj_r[...]
            fx = jnp.sum(wgt * (mi * (p0x - pxj_r[...]) + mj * (p0x - pxi_r[...])))
            fy = jnp.sum(wgt * (mi * (p0y - pyj_r[...]) + mj * (p0y - pyi_r[...])))
            fz = jnp.sum(wgt * (mi * (p0z - pzj_r[...]) + mj * (p0z - pzi_r[...])))
            rowid = lax.broadcasted_iota(jnp.int32, (8, 128), 0)
            colid = lax.broadcasted_iota(jnp.int32, (8, 128), 1)
            r0 = rowid == 0
            add = (jnp.where(r0 & (colid == 0), -fx, 0.0)
                   + jnp.where(r0 & (colid == 1), -fy, 0.0)
                   + jnp.where(r0 & (colid == 2), -fz, 0.0))
            o_ref[...] += add

    def bmap2(i, cnt_ref):
        return (jnp.where(i * BLKF < cnt_ref[0], i, 0), 0)

    def bmap1(i, cnt_ref):
        return (jnp.where(i * BLKF < cnt_ref[0], i, 0),)

    fspec = pl.BlockSpec((BLKF,), bmap1)
    grid_spec = pltpu.PrefetchScalarGridSpec(
        num_scalar_prefetch=1,
        grid=(EC // BLKF,),
        in_specs=[
            pl.BlockSpec((BLKF, D), bmap2),
            pl.BlockSpec((BLKF, D), bmap2),
            fspec, fspec, fspec, fspec, fspec, fspec, fspec, fspec, fspec,
            fspec,
            pl.BlockSpec((RB, D), lambda i, c: (0, 0)),
            pl.BlockSpec((D,), lambda i, c: (0,)),
            pl.BlockSpec((D, D), lambda i, c: (0, 0)),
            pl.BlockSpec((D,), lambda i, c: (0,)),
            pl.BlockSpec((RB, D), lambda i, c: (0, 0)),
            pl.BlockSpec((D,), lambda i, c: (0,)),
            pl.BlockSpec((D, D), lambda i, c: (0, 0)),
            pl.BlockSpec((D,), lambda i, c: (0,)),
            pl.BlockSpec((RB,), lambda i, c: (0,)),
            pl.BlockSpec((RB,), lambda i, c: (0,)),
            pl.BlockSpec((8, 128), lambda i, c: (0, 0)),
        ],
        out_specs=pl.BlockSpec((8, 128), lambda i, c: (0, 0)),
    )
    return pl.pallas_call(
        body,
        grid_spec=grid_spec,
        out_shape=jax.ShapeDtypeStruct((8, 128), jnp.float32),
        compiler_params=pltpu.CompilerParams(
            dimension_semantics=("arbitrary",)),
    )(cnt, df0, df1, *flds, w10, b10, w20, b20,
      w11, b11, w21, b21, centers, gamma, p0pad)


def kernel(positions, atom_types, idx_i, idx_j, seg_i, emb, centers, gamma,
           Wb1, bb1, Wc1, bc1, Wc2, bc2, Wb2, bb2, Wb3, bb3, Wd, bd, We):
    pos = positions.reshape(NAT, 3)
    types = atom_types.reshape(NAT).astype(jnp.int32)
    idx_i = idx_i.astype(jnp.int32)
    idx_j = idx_j.astype(jnp.int32)
    seg_i = seg_i.astype(jnp.int32)
    padg = jnp.zeros((PADE,), jnp.int32)
    padn = jnp.full((PADE,), NAT, jnp.int32)
    ii0 = jnp.concatenate([idx_i, padg])
    jj0 = jnp.concatenate([idx_j, padg])
    jjN = jnp.concatenate([idx_j, padn])
    sgN = jnp.concatenate([seg_i, padn])
    embp = jnp.pad(emb, ((0, 16 - emb.shape[0]), (0, 0)))
    p0pad = jnp.zeros((8, 128), jnp.float32).at[0, :3].set(pos[0])

    ss, counts, cii, cjj, css = _sc_distcomp(pos, ii0, jj0)
    feats0, x0 = _tc_embed(types, embp, Wb1[0], bb1[0])
    filt0 = _tc_filt(ss, Wc1[0], bc1[0], Wc2[0], bc2[0], centers, gamma)
    aggp0 = _sc_gms(x0, jj0, sgN, filt0)
    feats1, hpre0, x1 = _tc_mid0(aggp0, feats0, Wb2[0], bb2[0],
                                 Wb3[0], bb3[0], Wb1[1], bb1[1])
    filt1 = _tc_filt(ss, Wc1[1], bc1[1], Wc2[1], bc2[1], centers, gamma)
    aggp1 = _sc_gms(x1, jj0, sgN, filt1)
    feats2, hpre1 = _tc_mid1(aggp1, feats1, Wb2[1], bb2[1], Wb3[1], bb3[1])
    dfeats2, dagg1 = _tc_bwd1(feats2, hpre1, Wd, bd, We, Wb3[1], Wb2[1])
    dxp = _sc_gms(dagg1, ii0, jjN, filt1)
    dagg0 = _tc_bwd0(dfeats2, dxp, Wb1[1], hpre0, Wb2[0], Wb3[0])
    cnt, df0, df1, *flds = _sc_gats(counts, cii, cjj, css, pos,
                                    x0, x1, dagg0, dagg1)
    out = _tc_force(cnt, df0, df1, flds,
                    Wc1[0], bc1[0], Wc2[0], bc2[0],
                    Wc1[1], bc1[1], Wc2[1], bc2[1],
                    centers, gamma, p0pad)
    return out[0, :3].reshape(1, 1, 3)


# speedup vs baseline: 4.9556x; 1.4281x over previous
import functools
import jax
import jax.numpy as jnp
from jax import lax
from jax.experimental import pallas as pl
import jax.experimental.pallas.tpu as pltpu
from jax.experimental.pallas import tpu_sc as plsc

NAT = 10000
NE = 160000
D = 128
RB = 32
NW = 32
EW = 5008
E2 = NW * EW
PADE = E2 - NE
CAP = 5120
EC = NW * CAP
CH = 128
ZR = 624
NROW = NAT // 16
BLKN = 2000
BLKE = 2048
BLKF = 4096
LOG2 = 0.6931471805599453


def _ssp(x):
    return jnp.logaddexp(x, 0.0) - LOG2


def _sig(x):
    return jax.nn.sigmoid(x)


def _mesh():
    return plsc.VectorSubcoreMesh(core_axis_name="c", subcore_axis_name="s")


def _sc_distcomp(pos, ii0, jj0):
    def body(pos_hbm, ii_hbm, jj_hbm,
             ss_hbm, counts_hbm, cii_hbm, cjj_hbm, css_hbm,
             pos_v, ii_v, jj_v, ss_v, cii_v, cjj_v, css_v, sem):
        ci = lax.axis_index("c")
        si = lax.axis_index("s")
        w = ci * 16 + si
        base = w * EW
        pltpu.sync_copy(pos_hbm, pos_v)
        pltpu.sync_copy(ii_hbm.at[pl.ds(base, EW)], ii_v)
        pltpu.sync_copy(jj_hbm.at[pl.ds(base, EW)], jj_v)
        lanes = lax.broadcasted_iota(jnp.int32, (16,), 0)
        c1 = jnp.full((16,), 1, jnp.int32)
        c2 = jnp.full((16,), 2, jnp.int32)

        def step(t, cnt):
            ii = ii_v[pl.ds(t * 16, 16)]
            jj = jj_v[pl.ds(t * 16, 16)]
            fi = ii * 3
            fj = jj * 3
            xi = plsc.load_gather(pos_v, [fi])
            yi = plsc.load_gather(pos_v, [fi + c1])
            zi = plsc.load_gather(pos_v, [fi + c2])
            xj = plsc.load_gather(pos_v, [fj])
            yj = plsc.load_gather(pos_v, [fj + c1])
            zj = plsc.load_gather(pos_v, [fj + c2])
            dx = xi - xj
            dy = yi - yj
            dz = zi - zj
            ss = dx * dx + dy * dy + dz * dz
            ss_v[pl.ds(t * 16, 16)] = ss
            gid = base + t * 16 + lanes
            m = ((ii == 0) | (jj == 0)) & (gid < NE)
            plsc.store_compressed(cii_v.at[pl.ds(cnt, 16)], ii, mask=m)
            plsc.store_compressed(cjj_v.at[pl.ds(cnt, 16)], jj, mask=m)
            plsc.store_compressed(css_v.at[pl.ds(cnt, 16)], ss, mask=m)
            npos = jnp.max(plsc.all_reduce_population_count(m))
            return cnt + npos

        cnt = lax.fori_loop(0, EW // 16, step, jnp.int32(0))
        cii_v[pl.ds(cnt, 16)] = jnp.zeros((16,), jnp.int32)
        cjj_v[pl.ds(cnt, 16)] = jnp.zeros((16,), jnp.int32)
        css_v[pl.ds(cnt, 16)] = jnp.zeros((16,), jnp.float32)
        pltpu.sync_copy(ss_v, ss_hbm.at[pl.ds(base, EW)])
        ii_v[pl.ds(0, 16)] = c1 * cnt
        pltpu.sync_copy(ii_v.at[pl.ds(0, 16)],
                        counts_hbm.at[pl.ds(w * 16, 16)])
        wb = pl.multiple_of(w * CAP, 16)
        pltpu.sync_copy(cii_v, cii_hbm.at[pl.ds(wb, CAP)])
        pltpu.sync_copy(cjj_v, cjj_hbm.at[pl.ds(wb, CAP)])
        pltpu.sync_copy(css_v, css_hbm.at[pl.ds(wb, CAP)])

    f = pl.kernel(
        body,
        out_type=[
            jax.ShapeDtypeStruct((E2,), jnp.float32),
            jax.ShapeDtypeStruct((NW * 16,), jnp.int32),
            jax.ShapeDtypeStruct((NW * CAP,), jnp.int32),
            jax.ShapeDtypeStruct((NW * CAP,), jnp.int32),
            jax.ShapeDtypeStruct((NW * CAP,), jnp.float32),
        ],
        mesh=_mesh(),
        compiler_params=pltpu.CompilerParams(needs_layout_passes=False),
        scratch_types=[
            pltpu.VMEM((NAT * 3,), jnp.float32),
            pltpu.VMEM((EW,), jnp.int32),
            pltpu.VMEM((EW,), jnp.int32),
            pltpu.VMEM((EW,), jnp.float32),
            pltpu.VMEM((CAP,), jnp.int32),
            pltpu.VMEM((CAP,), jnp.int32),
            pltpu.VMEM((CAP,), jnp.float32),
            pltpu.SemaphoreType.DMA,
        ],
    )
    return f(pos.reshape(NAT * 3), ii0, jj0)


def _sc_gms(tab, gidx, sidx, filt):
    def body(tab_hbm, gi_hbm, si_hbm, filt_hbm, aggp_hbm,
             gi_v, si_v, gi_t, si_t, rows_v, filt_v, agg_sh, sem):
        ci = lax.axis_index("c")
        si_ax = lax.axis_index("s")
        w = ci * 16 + si_ax
        base = w * EW

        def zstep(i, _):
            r = lax.shift_right_logical(i, 3)
            k = (i & 7) * 16
            rows_v[r, pl.ds(k, 16)] = jnp.zeros((16,), jnp.float32)
            return 0

        lax.fori_loop(0, CH * 8, zstep, 0)
        r0 = pl.multiple_of(si_ax * ZR, 16)
        pltpu.sync_copy(rows_v, agg_sh.at[pl.ds(r0, CH), :])
        pltpu.sync_copy(rows_v, agg_sh.at[pl.ds(r0 + CH, CH), :])
        pltpu.sync_copy(rows_v, agg_sh.at[pl.ds(r0 + 2 * CH, CH), :])
        pltpu.sync_copy(rows_v, agg_sh.at[pl.ds(r0 + 3 * CH, CH), :])
        pltpu.sync_copy(rows_v.at[pl.ds(0, 112), :],
                        agg_sh.at[pl.ds(r0 + 4 * CH, 112), :])
        pltpu.sync_copy(rows_v.at[pl.ds(0, 16), :],
                        agg_sh.at[pl.ds(16 * ZR, 16), :])
        plsc.subcore_barrier()

        def mul_loop(nrows, rv, fv):
            def mstep(r, _):
                for k in range(8):
                    rv[r, pl.ds(k * 16, 16)] = (rv[r, pl.ds(k * 16, 16)]
                                                * fv[r, pl.ds(k * 16, 16)])
                return 0
            lax.fori_loop(0, nrows, mstep, 0)

        def chunk(t, _):
            bc = pl.multiple_of(base + t * CH, 16)
            pltpu.sync_copy(gi_hbm.at[pl.ds(bc, CH)], gi_v)
            pltpu.sync_copy(si_hbm.at[pl.ds(bc, CH)], si_v)
            pltpu.async_copy(tab_hbm.at[gi_v], rows_v, sem).wait()
            pltpu.sync_copy(filt_hbm.at[pl.ds(bc, CH), :], filt_v)
            mul_loop(CH, rows_v, filt_v)
            pltpu.sync_copy(rows_v, agg_sh.at[si_v], add=True)
            return 0

        lax.fori_loop(0, 39, chunk, 0)
        bt = base + 39 * CH
        pltpu.sync_copy(gi_hbm.at[pl.ds(bt, 16)], gi_t)
        pltpu.sync_copy(si_hbm.at[pl.ds(bt, 16)], si_t)
        pltpu.async_copy(tab_hbm.at[gi_t], rows_v.at[pl.ds(0, 16), :],
                         sem).wait()
        pltpu.sync_copy(filt_hbm.at[pl.ds(bt, 16), :],
                        filt_v.at[pl.ds(0, 16), :])
        mul_loop(16, rows_v, filt_v)
        pltpu.sync_copy(rows_v.at[pl.ds(0, 16), :], agg_sh.at[si_t], add=True)
        plsc.subcore_barrier()
        pltpu.sync_copy(agg_sh.at[pl.ds(r0, CH), :],
                        aggp_hbm.at[ci, pl.ds(r0, CH), :])
        pltpu.sync_copy(agg_sh.at[pl.ds(r0 + CH, CH), :],
                        aggp_hbm.at[ci, pl.ds(r0 + CH, CH), :])
        pltpu.sync_copy(agg_sh.at[pl.ds(r0 + 2 * CH, CH), :],
                        aggp_hbm.at[ci, pl.ds(r0 + 2 * CH, CH), :])
        pltpu.sync_copy(agg_sh.at[pl.ds(r0 + 3 * CH, CH), :],
                        aggp_hbm.at[ci, pl.ds(r0 + 3 * CH, CH), :])
        pltpu.sync_copy(agg_sh.at[pl.ds(r0 + 4 * CH, 112), :],
                        aggp_hbm.at[ci, pl.ds(r0 + 4 * CH, 112), :])
        pltpu.sync_copy(agg_sh.at[pl.ds(16 * ZR, 16), :],
                        aggp_hbm.at[ci, pl.ds(16 * ZR, 16), :])

    f = pl.kernel(
        body,
        out_type=jax.ShapeDtypeStruct((2, NAT, D), jnp.float32),
        mesh=_mesh(),
        compiler_params=pltpu.CompilerParams(needs_layout_passes=False),
        scratch_types=[
            pltpu.VMEM((CH,), jnp.int32),
            pltpu.VMEM((CH,), jnp.int32),
            pltpu.VMEM((16,), jnp.int32),
            pltpu.VMEM((16,), jnp.int32),
            pltpu.VMEM((CH, D), jnp.float32),
            pltpu.VMEM((CH, D), jnp.float32),
            pltpu.VMEM_SHARED((NAT + 16, D), jnp.float32),
            pltpu.SemaphoreType.DMA,
        ],
    )
    return f(tab, gidx, sidx, filt)


def _sc_gats(counts, cii, cjj, css, pos, x0, x1, dg0, dg1):
    def body(counts_hbm, cii_hbm, cjj_hbm, css_hbm, pos_hbm,
             x0_hbm, x1_hbm, dg0_hbm, dg1_hbm,
             cnt_hbm, df0_hbm, df1_hbm,
             pxi_hbm, pyi_hbm, pzi_hbm, pxj_hbm, pyj_hbm, pzj_hbm,
             ssv_hbm, mi_hbm, mj_hbm, vv_hbm,
             counts_v, pos_v, ii_t, jj_t, ssv_t,
             gx0, gx1, gd0, gd1, d0_v, d1_v, stage_v, stage_f, sem):
        ci = lax.axis_index("c")
        si = lax.axis_index("s")
        w = ci * 16 + si
        pltpu.sync_copy(counts_hbm, counts_v)
        pltpu.sync_copy(pos_hbm, pos_v)
        lanes = lax.broadcasted_iota(jnp.int32, (16,), 0)
        c1 = jnp.full((16,), 1, jnp.int32)
        c2 = jnp.full((16,), 2, jnp.int32)

        def acc(v, carry):
            off, mycnt, tot = carry
            cv = jnp.max(counts_v[pl.ds(v * 16, 16)])
            cp = (cv + 15) & (-16)
            off = off + jnp.where(v < w, cp, 0)
            mycnt = jnp.where(v == w, cv, mycnt)
            return (off, mycnt, tot + cp)

        off, mycnt, tot = lax.fori_loop(
            0, NW, acc, (jnp.int32(0), jnp.int32(0), jnp.int32(0)))
        stage_v[pl.ds(0, 16)] = jnp.full((16,), 1, jnp.int32) * tot
        pltpu.sync_copy(stage_v, cnt_hbm)
        nst = lax.shift_right_logical(mycnt + 15, 4)

        def fout(val, dst, orow):
            stage_f[pl.ds(0, 16)] = val
            pltpu.sync_copy(stage_f, dst.at[pl.ds(orow, 16)])

        def step(t, _):
            rb = pl.multiple_of(w * CAP + t * 16, 16)
            pltpu.sync_copy(cii_hbm.at[pl.ds(rb, 16)], ii_t)
            pltpu.sync_copy(cjj_hbm.at[pl.ds(rb, 16)], jj_t)
            pltpu.sync_copy(css_hbm.at[pl.ds(rb, 16)], ssv_t)
            pltpu.async_copy(x0_hbm.at[jj_t], gx0, sem).wait()
            pltpu.async_copy(x1_hbm.at[jj_t], gx1, sem).wait()
            pltpu.async_copy(dg0_hbm.at[ii_t], gd0, sem).wait()
            pltpu.async_copy(dg1_hbm.at[ii_t], gd1, sem).wait()

            def mstep(i, _):
                r = lax.shift_right_logical(i, 3)
                k = (i & 7) * 16
                d0_v[r, pl.ds(k, 16)] = (gx0[r, pl.ds(k, 16)]
                                         * gd0[r, pl.ds(k, 16)])
                d1_v[r, pl.ds(k, 16)] = (gx1[r, pl.ds(k, 16)]
                                         * gd1[r, pl.ds(k, 16)])
                return 0

            lax.fori_loop(0, 128, mstep, 0)
            orow = pl.multiple_of(off + t * 16, 16)
            pltpu.sync_copy(d0_v, df0_hbm.at[pl.ds(orow, 16), :])
            pltpu.sync_copy(d1_v, df1_hbm.at[pl.ds(orow, 16), :])
            ii = ii_t[...]
            jj = jj_t[...]
            one = jnp.full((16,), 1.0, jnp.float32)
            zero = jnp.zeros((16,), jnp.float32)
            fi = ii * 3
            fj = jj * 3
            fout(plsc.load_gather(pos_v, [fi]), pxi_hbm, orow)
            fout(plsc.load_gather(pos_v, [fi + c1]), pyi_hbm, orow)
            fout(plsc.load_gather(pos_v, [fi + c2]), pzi_hbm, orow)
            fout(plsc.load_gather(pos_v, [fj]), pxj_hbm, orow)
            fout(plsc.load_gather(pos_v, [fj + c1]), pyj_hbm, orow)
            fout(plsc.load_gather(pos_v, [fj + c2]), pzj_hbm, orow)
            fout(ssv_t[...], ssv_hbm, orow)
            fout(jnp.where(ii == 0, one, zero), mi_hbm, orow)
            fout(jnp.where(jj == 0, one, zero), mj_hbm, orow)
            fout(jnp.where(t * 16 + lanes < mycnt, one, zero), vv_hbm, orow)
            return 0

        lax.fori_loop(0, nst, step, 0)

    fld = jax.ShapeDtypeStruct((EC,), jnp.float32)
    f = pl.kernel(
        body,
        out_type=[
            jax.ShapeDtypeStruct((16,), jnp.int32),
            jax.ShapeDtypeStruct((EC, D), jnp.float32),
            jax.ShapeDtypeStruct((EC, D), jnp.float32),
            fld, fld, fld, fld, fld, fld, fld, fld, fld, fld,
        ],
        mesh=_mesh(),
        compiler_params=pltpu.CompilerParams(needs_layout_passes=False),
        scratch_types=[
            pltpu.VMEM((NW * 16,), jnp.int32),
            pltpu.VMEM((NAT * 3,), jnp.float32),
            pltpu.VMEM((16,), jnp.int32),
            pltpu.VMEM((16,), jnp.int32),
            pltpu.VMEM((16,), jnp.float32),
            pltpu.VMEM((16, D), jnp.float32),
            pltpu.VMEM((16, D), jnp.float32),
            pltpu.VMEM((16, D), jnp.float32),
            pltpu.VMEM((16, D), jnp.float32),
            pltpu.VMEM((16, D), jnp.float32),
            pltpu.VMEM((16, D), jnp.float32),
            pltpu.VMEM((16,), jnp.int32),
            pltpu.VMEM((16,), jnp.float32),
            pltpu.SemaphoreType.DMA,
        ],
    )
    return f(counts, cii, cjj, css, pos.reshape(NAT * 3), x0, x1, dg0, dg1)


def _tc_embed(types, embp, w1, b1):
    def body(t_ref, e_ref, w_ref, b_ref, f_ref, x_ref):
        oh = (t_ref[0, 0, :][:, None]
              == lax.broadcasted_iota(jnp.int32, (BLKN, 16), 1))
        f = oh.astype(jnp.float32) @ e_ref[...]
        f_ref[...] = f
        x_ref[...] = f @ w_ref[...] + b_ref[...][None, :]

    types = types.reshape(NAT // BLKN, 1, BLKN)
    return pl.pallas_call(
        body,
        grid=(NAT // BLKN,),
        in_specs=[
            pl.BlockSpec((1, 1, BLKN), lambda i: (i, 0, 0)),
            pl.BlockSpec((16, D), lambda i: (0, 0)),
            pl.BlockSpec((D, D), lambda i: (0, 0)),
            pl.BlockSpec((D,), lambda i: (0,)),
        ],
        out_specs=[
            pl.BlockSpec((BLKN, D), lambda i: (i, 0)),
            pl.BlockSpec((BLKN, D), lambda i: (i, 0)),
        ],
        out_shape=[
            jax.ShapeDtypeStruct((NAT, D), jnp.float32),
            jax.ShapeDtypeStruct((NAT, D), jnp.float32),
        ],
    )(types, embp, w1, b1)


def _tc_filt(ss, w1, b1, w2, b2, centers, gamma):
    def body(ss_ref, w1_ref, b1_ref, w2_ref, b2_ref, c_ref, g_ref, o_ref):
        dist = jnp.sqrt(ss_ref[...] + 1e-08)
        rbf = jnp.exp(-g_ref[...][None, :]
                      * (dist[:, None] - c_ref[...][None, :]) ** 2)
        a1 = rbf @ w1_ref[...] + b1_ref[...][None, :]
        a2 = _ssp(a1) @ w2_ref[...] + b2_ref[...][None, :]
        o_ref[...] = _ssp(a2)

    nblk = (E2 + BLKE - 1) // BLKE
    return pl.pallas_call(
        body,
        grid=(nblk,),
        in_specs=[
            pl.BlockSpec((BLKE,), lambda i: (i,)),
            pl.BlockSpec((RB, D), lambda i: (0, 0)),
            pl.BlockSpec((D,), lambda i: (0,)),
            pl.BlockSpec((D, D), lambda i: (0, 0)),
            pl.BlockSpec((D,), lambda i: (0,)),
            pl.BlockSpec((RB,), lambda i: (0,)),
            pl.BlockSpec((RB,), lambda i: (0,)),
        ],
        out_specs=pl.BlockSpec((BLKE, D), lambda i: (i, 0)),
        out_shape=jax.ShapeDtypeStruct((E2, D), jnp.float32),
    )(ss, w1, b1, w2, b2, centers, gamma)


def _node_specs(n_in, n_out, extra_w):
    in_specs = [pl.BlockSpec((BLKN, D), lambda i: (i, 0)) for _ in range(n_in)]
    in_specs += extra_w
    out_specs = [pl.BlockSpec((BLKN, D), lambda i: (i, 0))
                 for _ in range(n_out)]
    out_shape = [jax.ShapeDtypeStruct((NAT, D), jnp.float32)
                 for _ in range(n_out)]
    return in_specs, out_specs, out_shape


_WSPEC = pl.BlockSpec((D, D), lambda i: (0, 0))
_BSPEC = pl.BlockSpec((D,), lambda i: (0,))
_PSPEC = pl.BlockSpec((2, BLKN, D), lambda i: (0, i, 0))


def _tc_mid0(aggp, feats0, w2, b2, w3, b3, w1n, b1n):
    def body(a_ref, f_ref, w2r, b2r, w3r, b3r, w1r, b1r,
             f1_ref, hp_ref, x1_ref):
        agg = a_ref[0] + a_ref[1]
        hpre = agg @ w2r[...] + b2r[...][None, :]
        hp_ref[...] = hpre
        f1 = f_ref[...] + _ssp(hpre) @ w3r[...] + b3r[...][None, :]
        f1_ref[...] = f1
        x1_ref[...] = f1 @ w1r[...] + b1r[...][None, :]

    in_specs = [_PSPEC, pl.BlockSpec((BLKN, D), lambda i: (i, 0)),
                _WSPEC, _BSPEC, _WSPEC, _BSPEC, _WSPEC, _BSPEC]
    _, out_specs, out_shape = _node_specs(0, 3, [])
    return pl.pallas_call(
        body, grid=(NAT // BLKN,), in_specs=in_specs,
        out_specs=out_specs, out_shape=out_shape,
    )(aggp, feats0, w2, b2, w3, b3, w1n, b1n)


def _tc_mid1(aggp, feats1, w2, b2, w3, b3):
    def body(a_ref, f_ref, w2r, b2r, w3r, b3r, f2_ref, hp_ref):
        agg = a_ref[0] + a_ref[1]
        hpre = agg @ w2r[...] + b2r[...][None, :]
        hp_ref[...] = hpre
        f2_ref[...] = f_ref[...] + _ssp(hpre) @ w3r[...] + b3r[...][None, :]

    in_specs = [_PSPEC, pl.BlockSpec((BLKN, D), lambda i: (i, 0)),
                _WSPEC, _BSPEC, _WSPEC, _BSPEC]
    _, out_specs, out_shape = _node_specs(0, 2, [])
    return pl.pallas_call(
        body, grid=(NAT // BLKN,), in_specs=in_specs,
        out_specs=out_specs, out_shape=out_shape,
    )(aggp, feats1, w2, b2, w3, b3)


def _tc_bwd1(feats2, hpre1, Wd, bd, We, w3, w2):
    def body(f_ref, hp_ref, wd_ref, bd_ref, we_ref, w3r, w2r,
             df_ref, da_ref):
        z = f_ref[...] @ wd_ref[...] + bd_ref[...][None, :]
        df2 = lax.dot_general(
            _sig(z) * we_ref[...].reshape(1, D // 2), wd_ref[...],
            (((1,), (1,)), ((), ())))
        df_ref[...] = df2
        t = lax.dot_general(df2, w3r[...], (((1,), (1,)), ((), ())))
        da_ref[...] = lax.dot_general(
            t * _sig(hp_ref[...]), w2r[...], (((1,), (1,)), ((), ())))

    in_specs = [pl.BlockSpec((BLKN, D), lambda i: (i, 0)),
                pl.BlockSpec((BLKN, D), lambda i: (i, 0)),
                pl.BlockSpec((D, D // 2), lambda i: (0, 0)),
                pl.BlockSpec((D // 2,), lambda i: (0,)),
                pl.BlockSpec((D // 2, 1), lambda i: (0, 0)),
                _WSPEC, _WSPEC]
    _, out_specs, out_shape = _node_specs(0, 2, [])
    return pl.pallas_call(
        body, grid=(NAT // BLKN,), in_specs=in_specs,
        out_specs=out_specs, out_shape=out_shape,
    )(feats2, hpre1, Wd, bd, We, w3, w2)


def _tc_bwd0(dfeats2, dxp, w1n, hpre0, w2, w3):
    def body(df_ref, dx_ref, w1r, hp_ref, w2r, w3r, da_ref):
        dx1 = dx_ref[0] + dx_ref[1]
        df1 = df_ref[...] + lax.dot_general(
            dx1, w1r[...], (((1,), (1,)), ((), ())))
        t = lax.dot_general(df1, w3r[...], (((1,), (1,)), ((), ())))
        da_ref[...] = lax.dot_general(
            t * _sig(hp_ref[...]), w2r[...], (((1,), (1,)), ((), ())))

    in_specs = [pl.BlockSpec((BLKN, D), lambda i: (i, 0)), _PSPEC,
                _WSPEC, pl.BlockSpec((BLKN, D), lambda i: (i, 0)),
                _WSPEC, _WSPEC]
    _, out_specs, out_shape = _node_specs(0, 1, [])
    return pl.pallas_call(
        body, grid=(NAT // BLKN,), in_specs=in_specs,
        out_specs=out_specs, out_shape=out_shape,
    )(dfeats2, dxp, w1n, hpre0, w2, w3)[0]


def _tc_force(cnt, df0, df1, flds, w10, b10, w20, b20,
              w11, b11, w21, b21, centers, gamma, p0pad):
    def body(cnt_ref, df0_ref, df1_ref,
             pxi_r, pyi_r, pzi_r, pxj_r, pyj_r, pzj_r, ss_r, mi_r, mj_r, vv_r,
             w10r, b10r, w20r, b20r, w11r, b11r, w21r, b21r,
             c_ref, gm_ref, p0_ref, o_ref):
        i = pl.program_id(0)

        @pl.when(i == 0)
        def _init():
            o_ref[...] = jnp.zeros((8, 128), jnp.float32)

        @pl.when(i * BLKF < cnt_ref[0])
        def _work():
            ss = ss_r[...]
            dist = jnp.sqrt(ss + 1e-08)
            cen = c_ref[...][None, :]
            gam = gm_ref[...][None, :]
            diff = dist[:, None] - cen
            rbf = jnp.exp(-gam * diff ** 2)
            drbf = jnp.zeros((BLKF, RB), jnp.float32)
            for (w1r, b1r, w2r, b2r, dfr) in (
                    (w10r, b10r, w20r, b20r, df0_ref),
                    (w11r, b11r, w21r, b21r, df1_ref)):
                a1 = rbf @ w1r[...] + b1r[...][None, :]
                a2 = _ssp(a1) @ w2r[...] + b2r[...][None, :]
                da2 = dfr[...] * _sig(a2)
                da1 = lax.dot_general(
                    da2, w2r[...], (((1,), (1,)), ((), ()))) * _sig(a1)
                drbf = drbf + lax.dot_general(
                    da1, w1r[...], (((1,), (1,)), ((), ())))
            dd = jnp.sum(drbf * rbf * (-2.0 * gam * diff), axis=1)
            wgt = vv_r[...] * dd / dist
            p0x = p0_ref[0, 0]
            p0y = p0_ref[0, 1]
            p0z = p0_ref[0, 2]
            mi = mi_r[...]
            mj = mj_r[...]
            fx = jnp.sum(wgt * (mi * (p0x - pxj_r[...]) + mj * (p0x - pxi_r[...])))
            fy = jnp.sum(wgt * (mi * (p0y - pyj_r[...]) + mj * (p0y - pyi_r[...])))
            fz = jnp.sum(wgt * (mi * (p0z - pzj_r[...]) + mj * (p0z - pzi_r[...])))
            rowid = lax.broadcasted_iota(jnp.int32, (8, 128), 0)
            colid = lax.broadcasted_iota(jnp.int32, (8, 128), 1)
            r0 = rowid == 0
            add = (jnp.where(r0 & (colid == 0), -fx, 0.0)
                   + jnp.where(r0 & (colid == 1), -fy, 0.0)
                   + jnp.where(r0 & (colid == 2), -fz, 0.0))
            o_ref[...] += add

    def bmap2(i, cnt_ref):
        return (jnp.where(i * BLKF < cnt_ref[0], i, 0), 0)

    def bmap1(i, cnt_ref):
        return (jnp.where(i * BLKF < cnt_ref[0], i, 0),)

    fspec = pl.BlockSpec((BLKF,), bmap1)
    grid_spec = pltpu.PrefetchScalarGridSpec(
        num_scalar_prefetch=1,
        grid=(EC // BLKF,),
        in_specs=[
            pl.BlockSpec((BLKF, D), bmap2),
            pl.BlockSpec((BLKF, D), bmap2),
            fspec, fspec, fspec, fspec, fspec, fspec, fspec, fspec, fspec,
            fspec,
            pl.BlockSpec((RB, D), lambda i, c: (0, 0)),
            pl.BlockSpec((D,), lambda i, c: (0,)),
            pl.BlockSpec((D, D), lambda i, c: (0, 0)),
            pl.BlockSpec((D,), lambda i, c: (0,)),
            pl.BlockSpec((RB, D), lambda i, c: (0, 0)),
            pl.BlockSpec((D,), lambda i, c: (0,)),
            pl.BlockSpec((D, D), lambda i, c: (0, 0)),
            pl.BlockSpec((D,), lambda i, c: (0,)),
            pl.BlockSpec((RB,), lambda i, c: (0,)),
            pl.BlockSpec((RB,), lambda i, c: (0,)),
            pl.BlockSpec((8, 128), lambda i, c: (0, 0)),
        ],
        out_specs=pl.BlockSpec((8, 128), lambda i, c: (0, 0)),
    )
    return pl.pallas_call(
        body,
        grid_spec=grid_spec,
        out_shape=jax.ShapeDtypeStruct((8, 128), jnp.float32),
        compiler_params=pltpu.CompilerParams(
            dimension_semantics=("arbitrary",)),
    )(cnt, df0, df1, *flds, w10, b10, w20, b20,
      w11, b11, w21, b21, centers, gamma, p0pad)


def kernel(positions, atom_types, idx_i, idx_j, seg_i, emb, centers, gamma,
           Wb1, bb1, Wc1, bc1, Wc2, bc2, Wb2, bb2, Wb3, bb3, Wd, bd, We):
    pos = positions.reshape(NAT, 3)
    types = atom_types.reshape(NAT).astype(jnp.int32)
    idx_i = idx_i.astype(jnp.int32)
    idx_j = idx_j.astype(jnp.int32)
    seg_i = seg_i.astype(jnp.int32)
    padg = jnp.zeros((PADE,), jnp.int32)
    padn = jnp.full((PADE,), NAT, jnp.int32)
    ii0 = jnp.concatenate([idx_i, padg])
    jj0 = jnp.concatenate([idx_j, padg])
    jjN = jnp.concatenate([idx_j, padn])
    sgN = jnp.concatenate([seg_i, padn])
    embp = jnp.pad(emb, ((0, 16 - emb.shape[0]), (0, 0)))
    p0pad = jnp.zeros((8, 128), jnp.float32).at[0, :3].set(pos[0])

    ss, counts, cii, cjj, css = _sc_distcomp(pos, ii0, jj0)
    feats0, x0 = _tc_embed(types, embp, Wb1[0], bb1[0])
    filt0 = _tc_filt(ss, Wc1[0], bc1[0], Wc2[0], bc2[0], centers, gamma)
    aggp0 = _sc_gms(x0, jj0, sgN, filt0)
    feats1, hpre0, x1 = _tc_mid0(aggp0, feats0, Wb2[0], bb2[0],
                                 Wb3[0], bb3[0], Wb1[1], bb1[1])
    filt1 = _tc_filt(ss, Wc1[1], bc1[1], Wc2[1], bc2[1], centers, gamma)
    aggp1 = _sc_gms(x1, jj0, sgN, filt1)
    feats2, hpre1 = _tc_mid1(aggp1, feats1, Wb2[1], bb2[1], Wb3[1], bb3[1])
    dfeats2, dagg1 = _tc_bwd1(feats2, hpre1, Wd, bd, We, Wb3[1], Wb2[1])
    dxp = _sc_gms(dagg1, ii0, jjN, filt1)
    dagg0 = _tc_bwd0(dfeats2, dxp, Wb1[1], hpre0, Wb2[0], Wb3[0])
    cnt, df0, df1, *flds = _sc_gats(counts, cii, cjj, css, pos,
                                    x0, x1, dagg0, dagg1)
    out = _tc_force(cnt, df0, df1, flds,
                    Wc1[0], bc1[0], Wc2[0], bc2[0],
                    Wc1[1], bc1[1], Wc2[1], bc2[1],
                    centers, gamma, p0pad)
    return out[0, :3].reshape(1, 1, 3)
